# baseline (speedup 1.0000x reference)
import functools

import numpy as np
import jax
import jax.numpy as jnp
from jax import lax
from jax.experimental import pallas as pl
from jax.experimental.pallas import tpu as pltpu
from jax.experimental.pallas import tpu_sc as plsc

N = 10000
E = 160000
D = 16
H = 4
DH = D // H
NB = 16
HID = 64

NW = 32
CHUNK = 128
EPAD = 163840
EPW = EPAD // NW
NITER = EPW // CHUNK
SLAB = 8
NPAD = 10240
NPS = NPAD // 16
PW = 32

TE = 4096
TN = 1024

NC_A = 24
NC_B = NITER - NC_A
ROW_B = NW * NC_A


@functools.lru_cache(maxsize=None)
def _build_sc_kernels(row0, nc):
    epadh = NW * nc * CHUNK
    epwh = nc * CHUNK
    mesh = plsc.VectorSubcoreMesh(core_axis_name="c", subcore_axis_name="s")

    @functools.partial(
        pl.kernel,
        out_type=(jax.ShapeDtypeStruct((epadh, D), jnp.float32),
                  jax.ShapeDtypeStruct((epadh, D), jnp.float32)),
        mesh=mesh,
        scratch_types=[
            pltpu.VMEM((nc, CHUNK), jnp.int32),
            pltpu.VMEM((nc, CHUNK), jnp.int32),
            pltpu.VMEM((epwh, D), jnp.float32),
            pltpu.SemaphoreType.DMA,
        ],
        compiler_params=pltpu.CompilerParams(use_tc_tiling_on_sc=False),
    )
    def sc_gather(nf_hbm, src2_hbm, dst2_hbm, xs_hbm, xd_hbm,
                  idxs2, idxd2, rows, sem):
        c = lax.axis_index("c")
        s = lax.axis_index("s")
        wid = s * 2 + c
        pltpu.sync_copy(src2_hbm.at[pl.ds(row0 + wid * nc, nc)], idxs2)
        pltpu.sync_copy(dst2_hbm.at[pl.ds(row0 + wid * nc, nc)], idxd2)
        ebase = wid * epwh

        def one_pass(idx2, out_hbm):
            def body(b, carry):
                cps = []
                for j in range(SLAB):
                    k = b * SLAB + j
                    cps.append(pltpu.async_copy(
                        nf_hbm.at[idx2.at[k]],
                        rows.at[pl.ds(k * CHUNK, CHUNK)], sem))
                for cp in cps:
                    cp.wait()
                return carry

            lax.fori_loop(0, nc // SLAB, body, 0)
            pltpu.sync_copy(rows, out_hbm.at[pl.ds(ebase, epwh)])

        one_pass(idxs2, xs_hbm)
        one_pass(idxd2, xd_hbm)

    @functools.partial(
        pl.kernel,
        out_type=jax.ShapeDtypeStruct((2 * NPAD, PW), jnp.float32),
        mesh=mesh,
        scratch_types=[
            pltpu.VMEM((nc, CHUNK), jnp.int32),
            pltpu.VMEM((SLAB * CHUNK, PW), jnp.float32),
            pltpu.VMEM((NPS, PW), jnp.float32),
            pltpu.VMEM_SHARED((NPAD, PW), jnp.float32),
            pltpu.SemaphoreType.DMA,
        ],
        compiler_params=pltpu.CompilerParams(use_tc_tiling_on_sc=False),
    )
    def sc_aggregate(payload_hbm, dst2_hbm, zeros_hbm, out_hbm,
                     idxd2, pbuf, outv, acc_sh, sem):
        c = lax.axis_index("c")
        s = lax.axis_index("s")
        pltpu.sync_copy(zeros_hbm.at[pl.ds(s * NPS, NPS)],
                        acc_sh.at[pl.ds(s * NPS, NPS)])
        plsc.subcore_barrier()
        wid = s * 2 + c
        pltpu.sync_copy(dst2_hbm.at[pl.ds(row0 + wid * nc, nc)], idxd2)
        ebase = wid * epwh

        def body(b, carry):
            pltpu.sync_copy(
                payload_hbm.at[pl.ds(ebase + b * SLAB * CHUNK, SLAB * CHUNK)],
                pbuf)
            cps = []
            for j in range(SLAB):
                k = b * SLAB + j
                cps.append(pltpu.async_copy(
                    pbuf.at[pl.ds(j * CHUNK, CHUNK)],
                    acc_sh.at[idxd2.at[k]], sem, add=True))
            for cp in cps:
                cp.wait()
            return carry

        lax.fori_loop(0, nc // SLAB, body, 0)
        plsc.subcore_barrier()
        pltpu.sync_copy(acc_sh.at[pl.ds(s * NPS, NPS)], outv)
        pltpu.sync_copy(outv, out_hbm.at[pl.ds(c * NPAD + s * NPS, NPS)])

    return sc_gather, sc_aggregate


def _silu(x):
    return x * (1.0 / (1.0 + jnp.exp(-x)))


def _edge_body(r_ref, xs_ref, xd_ref, sh_ref,
               w1_ref, b1_ref, wk2_ref, bk2_ref, bv2_ref, wv2_ref,
               wqd_ref, rrep_ref, ssum_ref, shf_ref, out_ref):
    r = r_ref[...]
    h1 = _silu(r @ w1_ref[...] + b1_ref[...])
    hk = h1[:, :HID]
    hv = h1[:, HID:]
    kw = hk @ wk2_ref[...] + bk2_ref[...]
    vw = hv @ wv2_ref[...] + bv2_ref[...]
    xr = xs_ref[...] @ rrep_ref[...]
    kraw = (xr * kw) @ ssum_ref[...]
    vraw = (xr * vw) @ ssum_ref[...]
    qw = xd_ref[...] @ wqd_ref[...]
    sh = sh_ref[...]
    lg = ((qw * kraw) @ shf_ref[...]) * sh
    ex = jnp.exp(lg)
    vact = vraw * (sh * 0.25)
    exv = ex * vact
    out_ref[...] = jnp.concatenate([exv, ex], axis=1)


def _edge_stage(r, xs, xd, sh, w1, b1, wk2, bk2, bv2, wv2,
                wqd, rrep, ssum, shf, ntiles, tile0):
    off_spec = lambda w: pl.BlockSpec((TE, w), lambda i: (i + tile0, 0))
    edge_spec = lambda w: pl.BlockSpec((TE, w), lambda i: (i, 0))
    full_spec = lambda a, b: pl.BlockSpec((a, b), lambda i: (0, 0))
    return pl.pallas_call(
        _edge_body,
        grid=(ntiles,),
        in_specs=[
            off_spec(NB), edge_spec(D), edge_spec(D), off_spec(1),
            full_spec(NB, 2 * HID), full_spec(1, 2 * HID),
            full_spec(HID, D * D), full_spec(1, D * D),
            full_spec(1, D * D), full_spec(HID, D * D),
            full_spec(D, D), full_spec(D, D * D),
            full_spec(D * D, D), full_spec(D, D),
        ],
        out_specs=pl.BlockSpec((TE, PW), lambda i: (i, 0)),
        out_shape=jax.ShapeDtypeStruct((ntiles * TE, PW), jnp.float32),
    )(r, xs, xd, sh, w1, b1, wk2, bk2, bv2, wv2,
      wqd, rrep, ssum, shf)


def _final_body(nf_ref, p0_ref, p1_ref, p2_ref, p3_ref,
                wout_ref, wf1_ref, wf2_ref, out_ref):
    acc = (p0_ref[...] + p1_ref[...]) + (p2_ref[...] + p3_ref[...])
    num = acc[:, :D]
    den = acc[:, D:]
    out_h = num / (den + 1e-9)
    node = nf_ref[...] + out_h @ wout_ref[...]
    hq = _silu(node @ wf1_ref[...])
    out_ref[...] = node + hq @ wf2_ref[...]


def _final_stage(nf_pad, partsA, partsB, wout4, wf1s, wf2s):
    nb = NPAD // TN
    return pl.pallas_call(
        _final_body,
        grid=(nb,),
        in_specs=[
            pl.BlockSpec((TN, D), lambda i: (i, 0)),
            pl.BlockSpec((TN, PW), lambda i: (i, 0)),
            pl.BlockSpec((TN, PW), lambda i: (i + nb, 0)),
            pl.BlockSpec((TN, PW), lambda i: (i, 0)),
            pl.BlockSpec((TN, PW), lambda i: (i + nb, 0)),
            pl.BlockSpec((D, D), lambda i: (0, 0)),
            pl.BlockSpec((D, 2 * D), lambda i: (0, 0)),
            pl.BlockSpec((2 * D, D), lambda i: (0, 0)),
        ],
        out_specs=pl.BlockSpec((TN, D), lambda i: (i, 0)),
        out_shape=jax.ShapeDtypeStruct((NPAD, D), jnp.float32),
    )(nf_pad, partsA, partsA, partsB, partsB, wout4, wf1s, wf2s)


_RREP = np.kron(np.eye(D), np.ones((1, D))).astype(np.float32)
_SSUM = np.kron(np.ones((D, 1)), np.eye(D)).astype(np.float32)
_SHF = np.kron(np.eye(H), np.ones((DH, DH))).astype(np.float32)


def kernel(node_features, edge_index, edge_sh, edge_radial_emb,
           Wq, Wk1, bk1, Wk2, bk2, Wv1, bv1, Wv2, bv2, Wdot, Wout, Wf1, Wf2):
    f32 = jnp.float32
    src = jnp.concatenate(
        [edge_index[0].astype(jnp.int32),
         jnp.zeros((EPAD - E,), jnp.int32)])
    dst = jnp.concatenate(
        [edge_index[1].astype(jnp.int32),
         jnp.full((EPAD - E,), NPAD - 1, jnp.int32)])
    r = jnp.concatenate(
        [edge_radial_emb.astype(f32), jnp.zeros((EPAD - E, NB), f32)])
    sh = jnp.concatenate(
        [edge_sh.astype(f32), jnp.zeros((EPAD - E, 1), f32)])
    nf_pad = jnp.concatenate(
        [node_features.astype(f32), jnp.zeros((NPAD - N, D), f32)])

    wdot_bd = jnp.kron(jnp.eye(H, dtype=f32), Wdot.astype(f32))
    wqd = (Wq.astype(f32) @ wdot_bd) * (1.0 / 128.0)
    wout4 = Wout.astype(f32) * 0.25
    wf1s = Wf1.astype(f32) * 0.25
    wf2s = Wf2.astype(f32) * (1.0 / np.sqrt(2 * D))

    gather_a, agg_a = _build_sc_kernels(0, NC_A)
    gather_b, agg_b = _build_sc_kernels(ROW_B, NC_B)
    src2 = src.reshape(EPAD // CHUNK, CHUNK)
    dst2 = dst.reshape(EPAD // CHUNK, CHUNK)
    w1 = jnp.concatenate([Wk1.astype(f32), Wv1.astype(f32)], axis=1)
    b1 = jnp.concatenate([bk1.astype(f32), bv1.astype(f32)]).reshape(1, 2 * HID)
    edge_weights = (
        w1, b1,
        Wk2.astype(f32), bk2.astype(f32).reshape(1, D * D),
        bv2.astype(f32).reshape(1, D * D), Wv2.astype(f32),
        wqd, jnp.asarray(_RREP), jnp.asarray(_SSUM), jnp.asarray(_SHF))
    zeros = jnp.zeros((NPAD, PW), f32)

    tiles_a = NW * NC_A * CHUNK // TE
    tiles_b = NW * NC_B * CHUNK // TE
    xs_a, xd_a = gather_a(nf_pad, src2, dst2)
    payload_a = _edge_stage(r, xs_a, xd_a, sh, *edge_weights,
                            ntiles=tiles_a, tile0=0)
    xs_b, xd_b = gather_b(nf_pad, src2, dst2)
    payload_b = _edge_stage(r, xs_b, xd_b, sh, *edge_weights,
                            ntiles=tiles_b, tile0=tiles_a)
    parts_a = agg_a(payload_a, dst2, zeros)
    parts_b = agg_b(payload_b, dst2, zeros)
    out = _final_stage(nf_pad, parts_a, parts_b, wout4, wf1s, wf2s)
    return out[:N]


# speedup vs baseline: 1.0238x; 1.0238x over previous
import functools

import numpy as np
import jax
import jax.numpy as jnp
from jax import lax
from jax.experimental import pallas as pl
from jax.experimental.pallas import tpu as pltpu
from jax.experimental.pallas import tpu_sc as plsc

N = 10000
E = 160000
D = 16
H = 4
DH = D // H
NB = 16
HID = 64

NW = 32
CHUNK = 128
EPAD = 163840
EPW = EPAD // NW
NITER = EPW // CHUNK
SLAB = 8
NPAD = 10240
NPS = NPAD // 16
PW = 32

TE = 4096
TN = 1024

@functools.lru_cache(maxsize=None)
def _build_sc_kernels(row0, nc):
    epadh = NW * nc * CHUNK
    epwh = nc * CHUNK
    mesh = plsc.VectorSubcoreMesh(core_axis_name="c", subcore_axis_name="s")

    @functools.partial(
        pl.kernel,
        out_type=(jax.ShapeDtypeStruct((epadh, D), jnp.float32),
                  jax.ShapeDtypeStruct((epadh, D), jnp.float32)),
        mesh=mesh,
        scratch_types=[
            pltpu.VMEM((nc, CHUNK), jnp.int32),
            pltpu.VMEM((nc, CHUNK), jnp.int32),
            pltpu.VMEM((epwh, D), jnp.float32),
            pltpu.SemaphoreType.DMA,
        ],
        compiler_params=pltpu.CompilerParams(use_tc_tiling_on_sc=False),
    )
    def sc_gather(nf_hbm, src2_hbm, dst2_hbm, xs_hbm, xd_hbm,
                  idxs2, idxd2, rows, sem):
        c = lax.axis_index("c")
        s = lax.axis_index("s")
        wid = s * 2 + c
        pltpu.sync_copy(src2_hbm.at[pl.ds(row0 + wid * nc, nc)], idxs2)
        pltpu.sync_copy(dst2_hbm.at[pl.ds(row0 + wid * nc, nc)], idxd2)
        ebase = wid * epwh

        def one_pass(idx2, out_hbm):
            def body(b, carry):
                cps = []
                for j in range(SLAB):
                    k = b * SLAB + j
                    cps.append(pltpu.async_copy(
                        nf_hbm.at[idx2.at[k]],
                        rows.at[pl.ds(k * CHUNK, CHUNK)], sem))
                for cp in cps:
                    cp.wait()
                return carry

            lax.fori_loop(0, nc // SLAB, body, 0)
            pltpu.sync_copy(rows, out_hbm.at[pl.ds(ebase, epwh)])

        one_pass(idxs2, xs_hbm)
        one_pass(idxd2, xd_hbm)

    @functools.partial(
        pl.kernel,
        out_type=jax.ShapeDtypeStruct((2 * NPAD, PW), jnp.float32),
        mesh=mesh,
        scratch_types=[
            pltpu.VMEM((nc, CHUNK), jnp.int32),
            pltpu.VMEM((SLAB * CHUNK, PW), jnp.float32),
            pltpu.VMEM((NPS, PW), jnp.float32),
            pltpu.VMEM_SHARED((NPAD, PW), jnp.float32),
            pltpu.SemaphoreType.DMA,
        ],
        compiler_params=pltpu.CompilerParams(use_tc_tiling_on_sc=False),
    )
    def sc_aggregate(payload_hbm, dst2_hbm, zeros_hbm, out_hbm,
                     idxd2, pbuf, outv, acc_sh, sem):
        c = lax.axis_index("c")
        s = lax.axis_index("s")
        pltpu.sync_copy(zeros_hbm.at[pl.ds(s * NPS, NPS)],
                        acc_sh.at[pl.ds(s * NPS, NPS)])
        plsc.subcore_barrier()
        wid = s * 2 + c
        pltpu.sync_copy(dst2_hbm.at[pl.ds(row0 + wid * nc, nc)], idxd2)
        ebase = wid * epwh

        def body(b, carry):
            pltpu.sync_copy(
                payload_hbm.at[pl.ds(ebase + b * SLAB * CHUNK, SLAB * CHUNK)],
                pbuf)
            cps = []
            for j in range(SLAB):
                k = b * SLAB + j
                cps.append(pltpu.async_copy(
                    pbuf.at[pl.ds(j * CHUNK, CHUNK)],
                    acc_sh.at[idxd2.at[k]], sem, add=True))
            for cp in cps:
                cp.wait()
            return carry

        lax.fori_loop(0, nc // SLAB, body, 0)
        plsc.subcore_barrier()
        pltpu.sync_copy(acc_sh.at[pl.ds(s * NPS, NPS)], outv)
        pltpu.sync_copy(outv, out_hbm.at[pl.ds(c * NPAD + s * NPS, NPS)])

    return sc_gather, sc_aggregate


def _silu(x):
    return x * (1.0 / (1.0 + jnp.exp(-x)))


def _edge_body(r_ref, xs_ref, xd_ref, sh_ref,
               w1_ref, b1_ref, wk2_ref, bk2_ref, bv2_ref, wv2_ref,
               wqd_ref, rrep_ref, ssum_ref, shf_ref, out_ref):
    r = r_ref[...]
    h1 = _silu(r @ w1_ref[...] + b1_ref[...])
    hk = h1[:, :HID]
    hv = h1[:, HID:]
    kw = hk @ wk2_ref[...] + bk2_ref[...]
    vw = hv @ wv2_ref[...] + bv2_ref[...]
    xr = xs_ref[...] @ rrep_ref[...]
    kraw = (xr * kw) @ ssum_ref[...]
    vraw = (xr * vw) @ ssum_ref[...]
    qw = xd_ref[...] @ wqd_ref[...]
    sh = sh_ref[...]
    lg = ((qw * kraw) @ shf_ref[...]) * sh
    ex = jnp.exp(lg)
    vact = vraw * (sh * 0.25)
    exv = ex * vact
    out_ref[...] = jnp.concatenate([exv, ex], axis=1)


def _edge_stage(r, xs, xd, sh, w1, b1, wk2, bk2, bv2, wv2,
                wqd, rrep, ssum, shf, ntiles, tile0):
    off_spec = lambda w: pl.BlockSpec((TE, w), lambda i: (i + tile0, 0))
    edge_spec = lambda w: pl.BlockSpec((TE, w), lambda i: (i, 0))
    full_spec = lambda a, b: pl.BlockSpec((a, b), lambda i: (0, 0))
    return pl.pallas_call(
        _edge_body,
        grid=(ntiles,),
        in_specs=[
            off_spec(NB), edge_spec(D), edge_spec(D), off_spec(1),
            full_spec(NB, 2 * HID), full_spec(1, 2 * HID),
            full_spec(HID, D * D), full_spec(1, D * D),
            full_spec(1, D * D), full_spec(HID, D * D),
            full_spec(D, D), full_spec(D, D * D),
            full_spec(D * D, D), full_spec(D, D),
        ],
        out_specs=pl.BlockSpec((TE, PW), lambda i: (i, 0)),
        out_shape=jax.ShapeDtypeStruct((ntiles * TE, PW), jnp.float32),
    )(r, xs, xd, sh, w1, b1, wk2, bk2, bv2, wv2,
      wqd, rrep, ssum, shf)


def _final_body(nf_ref, p0_ref, p1_ref, wout_ref, wf1_ref, wf2_ref, out_ref):
    acc = p0_ref[...] + p1_ref[...]
    num = acc[:, :D]
    den = acc[:, D:]
    out_h = num / (den + 1e-9)
    node = nf_ref[...] + out_h @ wout_ref[...]
    hq = _silu(node @ wf1_ref[...])
    out_ref[...] = node + hq @ wf2_ref[...]


def _final_stage(nf_pad, parts, wout4, wf1s, wf2s):
    nb = NPAD // TN
    return pl.pallas_call(
        _final_body,
        grid=(nb,),
        in_specs=[
            pl.BlockSpec((TN, D), lambda i: (i, 0)),
            pl.BlockSpec((TN, PW), lambda i: (i, 0)),
            pl.BlockSpec((TN, PW), lambda i: (i + nb, 0)),
            pl.BlockSpec((D, D), lambda i: (0, 0)),
            pl.BlockSpec((D, 2 * D), lambda i: (0, 0)),
            pl.BlockSpec((2 * D, D), lambda i: (0, 0)),
        ],
        out_specs=pl.BlockSpec((TN, D), lambda i: (i, 0)),
        out_shape=jax.ShapeDtypeStruct((NPAD, D), jnp.float32),
    )(nf_pad, parts, parts, wout4, wf1s, wf2s)


_RREP = np.kron(np.eye(D), np.ones((1, D))).astype(np.float32)
_SSUM = np.kron(np.ones((D, 1)), np.eye(D)).astype(np.float32)
_SHF = np.kron(np.eye(H), np.ones((DH, DH))).astype(np.float32)


def kernel(node_features, edge_index, edge_sh, edge_radial_emb,
           Wq, Wk1, bk1, Wk2, bk2, Wv1, bv1, Wv2, bv2, Wdot, Wout, Wf1, Wf2):
    f32 = jnp.float32
    src = jnp.concatenate(
        [edge_index[0].astype(jnp.int32),
         jnp.zeros((EPAD - E,), jnp.int32)])
    dst = jnp.concatenate(
        [edge_index[1].astype(jnp.int32),
         jnp.full((EPAD - E,), NPAD - 1, jnp.int32)])
    r = jnp.concatenate(
        [edge_radial_emb.astype(f32), jnp.zeros((EPAD - E, NB), f32)])
    sh = jnp.concatenate(
        [edge_sh.astype(f32), jnp.zeros((EPAD - E, 1), f32)])
    nf_pad = jnp.concatenate(
        [node_features.astype(f32), jnp.zeros((NPAD - N, D), f32)])

    wdot_bd = jnp.kron(jnp.eye(H, dtype=f32), Wdot.astype(f32))
    wqd = (Wq.astype(f32) @ wdot_bd) * (1.0 / 128.0)
    wout4 = Wout.astype(f32) * 0.25
    wf1s = Wf1.astype(f32) * 0.25
    wf2s = Wf2.astype(f32) * (1.0 / np.sqrt(2 * D))

    sc_gather, sc_aggregate = _build_sc_kernels(0, NITER)
    src2 = src.reshape(EPAD // CHUNK, CHUNK)
    dst2 = dst.reshape(EPAD // CHUNK, CHUNK)
    w1 = jnp.concatenate([Wk1.astype(f32), Wv1.astype(f32)], axis=1)
    b1 = jnp.concatenate([bk1.astype(f32), bv1.astype(f32)]).reshape(1, 2 * HID)
    edge_weights = (
        w1, b1,
        Wk2.astype(f32), bk2.astype(f32).reshape(1, D * D),
        bv2.astype(f32).reshape(1, D * D), Wv2.astype(f32),
        wqd, jnp.asarray(_RREP), jnp.asarray(_SSUM), jnp.asarray(_SHF))
    zeros = jnp.zeros((NPAD, PW), f32)

    xs, xd = sc_gather(nf_pad, src2, dst2)
    payload = _edge_stage(r, xs, xd, sh, *edge_weights,
                          ntiles=EPAD // TE, tile0=0)
    parts = sc_aggregate(payload, dst2, zeros)
    out = _final_stage(nf_pad, parts, wout4, wf1s, wf2s)
    return out[:N]
